# baseline (speedup 1.0000x reference)
import functools

import jax
import jax.numpy as jnp
from jax import lax
from jax.experimental import pallas as pl
from jax.experimental.pallas import tpu as pltpu
from jax.experimental.pallas import tpu_sc as plsc

N = 10000
E = 320000
D = 128

NC = 2
NS = 16
L = 16
NW = NC * NS

K = 128
CPT = 80
CH = NW * CPT
EPAD = CH * K

NPAD = 10240
RPT = NPAD // NS

_mesh = plsc.VectorSubcoreMesh(
    core_axis_name="c", subcore_axis_name="s", num_cores=NC, num_subcores=NS)

_i32 = jnp.int32
_f32 = jnp.float32


def _deg_body(dst1, ew1, k11, k21, deg_out,
              h0, h1, h2, dst_v, ew_v, k1_v, k2_v, red_v, acc_v, stage_sh):
    c_ax = lax.axis_index("c")
    s_ax = lax.axis_index("s")
    wid = c_ax * NS + s_ax

    z16 = jnp.zeros((L,), _f32)

    def zero_body(i, _):
        h0[pl.ds(i * L, L)] = z16
        h1[pl.ds(i * L, L)] = z16
        h2[pl.ds(i * L, L)] = z16
        return 0
    lax.fori_loop(0, NPAD // L, zero_body, 0)

    def chunk_body(i, _):
        base = (wid * CPT + i) * K
        pltpu.sync_copy(dst1.at[pl.ds(base, K)], dst_v)
        pltpu.sync_copy(ew1.at[pl.ds(base, K)], ew_v)
        pltpu.sync_copy(k11.at[pl.ds(base, K)], k1_v)
        pltpu.sync_copy(k21.at[pl.ds(base, K)], k2_v)

        def grp(g, _):
            sl = pl.ds(g * L, L)
            d16 = dst_v[sl]
            w16 = ew_v[sl]
            plsc.addupdate_scatter(h0, [d16], w16)
            plsc.addupdate_scatter(h1, [d16], w16 * k1_v[sl])
            plsc.addupdate_scatter(h2, [d16], w16 * k2_v[sl])
            return 0
        lax.fori_loop(0, K // L, grp, 0)
        return 0
    lax.fori_loop(0, CPT, chunk_body, 0)

    pltpu.sync_copy(h0, stage_sh.at[pl.ds((s_ax * 3 + 0) * NPAD, NPAD)])
    pltpu.sync_copy(h1, stage_sh.at[pl.ds((s_ax * 3 + 1) * NPAD, NPAD)])
    pltpu.sync_copy(h2, stage_sh.at[pl.ds((s_ax * 3 + 2) * NPAD, NPAD)])
    plsc.subcore_barrier()

    for v in range(3):
        pltpu.sync_copy(stage_sh.at[pl.ds(v * NPAD + s_ax * RPT, RPT)], acc_v)

        def red_t(t, _):
            pltpu.sync_copy(
                stage_sh.at[pl.ds((t * 3 + v) * NPAD + s_ax * RPT, RPT)], red_v)

            def addg(g, _):
                sl = pl.ds(g * L, L)
                acc_v[sl] = acc_v[sl] + red_v[sl]
                return 0
            lax.fori_loop(0, RPT // L, addg, 0)
            return 0
        lax.fori_loop(1, NS, red_t, 0)
        pltpu.sync_copy(
            acc_v,
            deg_out.at[pl.ds((c_ax * 3 + v) * NPAD + s_ax * RPT, RPT)])


_deg_kernel = functools.partial(
    pl.kernel,
    out_type=jax.ShapeDtypeStruct((NC * 3 * NPAD,), _f32),
    mesh=_mesh,
    compiler_params=pltpu.CompilerParams(needs_layout_passes=False),
    scratch_types=[
        pltpu.VMEM((NPAD,), _f32),
        pltpu.VMEM((NPAD,), _f32),
        pltpu.VMEM((NPAD,), _f32),
        pltpu.VMEM((K,), _i32),
        pltpu.VMEM((K,), _f32),
        pltpu.VMEM((K,), _f32),
        pltpu.VMEM((K,), _f32),
        pltpu.VMEM((RPT,), _f32),
        pltpu.VMEM((RPT,), _f32),
        pltpu.VMEM_SHARED((NS * 3 * NPAD,), _f32),
    ],
)(_deg_body)


def _prop_body(table, src1, dst1, ew1, keep1d, dinv_hbm, part_out,
               dinv_v, srcA, dstA, ewA, keA, cA, rowsA,
               srcB, dstB, ewB, keB, cB, rowsB,
               acc_sh, esemA, esemB, gsemA, gsemB, ssemA, ssemB):
    c_ax = lax.axis_index("c")
    s_ax = lax.axis_index("s")
    wid = c_ax * NS + s_ax

    pltpu.sync_copy(dinv_hbm, dinv_v)

    z16 = jnp.zeros((L,), _f32)
    zi16 = jnp.zeros((L,), _i32)

    def zrow(k, _):
        for jj in range(D // L):
            rowsA[k, pl.ds(jj * L, L)] = z16
        return 0
    lax.fori_loop(0, K, zrow, 0)
    for q in range(RPT // K):
        pltpu.sync_copy(rowsA, acc_sh.at[pl.ds(s_ax * RPT + q * K, K)])
    plsc.subcore_barrier()

    cbase = wid * CPT

    def edge_dma(chunk, sv, dv, wv, kv, sem):
        eb = chunk * K
        pltpu.async_copy(src1.at[pl.ds(eb, K)], sv, sem)
        pltpu.async_copy(dst1.at[pl.ds(eb, K)], dv, sem)
        pltpu.async_copy(ew1.at[pl.ds(eb, K)], wv, sem)
        pltpu.async_copy(keep1d.at[pl.ds(eb, K)], kv, sem)

    def edge_wait(sv, dv, wv, kv, sem):
        pltpu.make_async_copy(src1.at[pl.ds(0, K)], sv, sem).wait()
        pltpu.make_async_copy(dst1.at[pl.ds(0, K)], dv, sem).wait()
        pltpu.make_async_copy(ew1.at[pl.ds(0, K)], wv, sem).wait()
        pltpu.make_async_copy(keep1d.at[pl.ds(0, K)], kv, sem).wait()

    def compute(sv, wv, kv, cv, rows):
        def cgrp(g, _):
            sl = pl.ds(g * L, L)
            dvv = plsc.load_gather(dinv_v, [sv[sl]])
            cv[sl] = wv[sl] * kv[sl] * dvv
            return 0
        lax.fori_loop(0, K // L, cgrp, 0)

        def scale(k, _):
            cb = plsc.load_gather(cv, [zi16 + k])
            for jj in range(D // L):
                sl = pl.ds(jj * L, L)
                rows[k, sl] = rows[k, sl] * cb
            return 0
        lax.fori_loop(0, K, scale, 0)

    edge_dma(cbase, srcA, dstA, ewA, keA, esemA)
    edge_wait(srcA, dstA, ewA, keA, esemA)
    pltpu.async_copy(table.at[srcA], rowsA, gsemA)

    def half(g, first, setX, setY):
        (srcX, dstX, ewX, keX, cX, rowsX, esemX, gsemX, ssemX) = setX
        (srcY, dstY, ewY, keY, cY, rowsY, esemY, gsemY, ssemY) = setY
        nxt = jnp.minimum(g + 1, CPT - 1)
        pltpu.make_async_copy(table.at[srcX], rowsX, gsemX).wait()
        if first:
            @pl.when(g > 0)
            def _():
                pltpu.make_async_copy(rowsY, acc_sh.at[dstY], ssemY).wait()
        else:
            pltpu.make_async_copy(rowsY, acc_sh.at[dstY], ssemY).wait()
        edge_dma(cbase + nxt, srcY, dstY, ewY, keY, esemY)
        compute(srcX, ewX, keX, cX, rowsX)
        edge_wait(srcY, dstY, ewY, keY, esemY)
        pltpu.async_copy(table.at[srcY], rowsY, gsemY)
        pltpu.async_copy(rowsX, acc_sh.at[dstX], ssemX, add=True)

    A = (srcA, dstA, ewA, keA, cA, rowsA, esemA, gsemA, ssemA)
    B = (srcB, dstB, ewB, keB, cB, rowsB, esemB, gsemB, ssemB)

    def body(gg, _):
        half(2 * gg, True, A, B)
        half(2 * gg + 1, False, B, A)
        return 0
    lax.fori_loop(0, CPT // 2, body, 0)

    pltpu.make_async_copy(table.at[srcA], rowsA, gsemA).wait()
    pltpu.make_async_copy(rowsB, acc_sh.at[dstB], ssemB).wait()
    plsc.subcore_barrier()

    for q in range(RPT // K):
        base = s_ax * RPT + q * K
        pltpu.sync_copy(acc_sh.at[pl.ds(base, K)], rowsA)

        def wscale(k, _):
            db = plsc.load_gather(dinv_v, [zi16 + base + k])
            for jj in range(D // L):
                sl = pl.ds(jj * L, L)
                rowsA[k, sl] = rowsA[k, sl] * db
            return 0
        lax.fori_loop(0, K, wscale, 0)
        pltpu.sync_copy(rowsA, part_out.at[pl.ds(c_ax * NPAD + base, K)])


_prop_kernel = functools.partial(
    pl.kernel,
    out_type=jax.ShapeDtypeStruct((NC * NPAD, D), _f32),
    mesh=_mesh,
    compiler_params=pltpu.CompilerParams(needs_layout_passes=False),
    scratch_types=[
        pltpu.VMEM((NPAD,), _f32),
        pltpu.VMEM((K,), _i32),
        pltpu.VMEM((K,), _i32),
        pltpu.VMEM((K,), _f32),
        pltpu.VMEM((K,), _f32),
        pltpu.VMEM((K,), _f32),
        pltpu.VMEM((K, D), _f32),
        pltpu.VMEM((K,), _i32),
        pltpu.VMEM((K,), _i32),
        pltpu.VMEM((K,), _f32),
        pltpu.VMEM((K,), _f32),
        pltpu.VMEM((K,), _f32),
        pltpu.VMEM((K, D), _f32),
        pltpu.VMEM_SHARED((NPAD, D), _f32),
        pltpu.SemaphoreType.DMA,
        pltpu.SemaphoreType.DMA,
        pltpu.SemaphoreType.DMA,
        pltpu.SemaphoreType.DMA,
        pltpu.SemaphoreType.DMA,
        pltpu.SemaphoreType.DMA,
    ],
)(_prop_body)


def _prep_body(deg_ref, dinv_ref):
    a = deg_ref[...]
    dinv_ref[...] = lax.rsqrt(a[0:3] + a[3:6] + 1e-9)


def _tc_prep(degparts):
    return pl.pallas_call(
        _prep_body,
        out_shape=jax.ShapeDtypeStruct((3, NPAD), _f32),
    )(degparts)


_BR = 512


def _dense_body(p0_ref, p1_ref, fm_ref, w_ref, b_ref, o_ref, *, relu):
    a = p0_ref[...] + p1_ref[...]
    a = a * fm_ref[...]
    o = jnp.dot(a, w_ref[...], preferred_element_type=_f32) + b_ref[...]
    if relu:
        o = jnp.maximum(o, 0.0)
    o_ref[...] = o


def _tc_dense(parts, fm, W, b, relu):
    body = functools.partial(_dense_body, relu=relu)
    nb = NPAD // _BR
    return pl.pallas_call(
        body,
        grid=(nb,),
        in_specs=[
            pl.BlockSpec((_BR, D), lambda i: (i, 0)),
            pl.BlockSpec((_BR, D), lambda i, nb=nb: (nb + i, 0)),
            pl.BlockSpec((1, D), lambda i: (0, 0)),
            pl.BlockSpec((D, D), lambda i: (0, 0)),
            pl.BlockSpec((1, D), lambda i: (0, 0)),
        ],
        out_specs=pl.BlockSpec((_BR, D), lambda i: (i, 0)),
        out_shape=jax.ShapeDtypeStruct((NPAD, D), _f32),
    )(parts, parts, fm, W, b)


def _pad1d(a, fill):
    pad = EPAD - E
    return jnp.concatenate([a, jnp.full((pad,), fill, a.dtype)])


def kernel(x, edge_index, edge_weight, W1, b1, W2, b2):
    k = jax.random.key(42)
    k1, k2, k3, k4 = jax.random.split(k, 4)
    keep1 = jax.random.bernoulli(k1, 0.7, (E,)).astype(_f32)
    keep2 = jax.random.bernoulli(k2, 0.7, (E,)).astype(_f32)
    fm1 = jax.random.bernoulli(k3, 0.7, (D,)).astype(_f32)
    fm2 = jax.random.bernoulli(k4, 0.7, (D,)).astype(_f32)

    src1 = _pad1d(edge_index[0], 0)
    dst1 = _pad1d(edge_index[1], N)
    ew1 = _pad1d(edge_weight, 0.0)
    ones1 = _pad1d(jnp.ones((E,), _f32), 0.0)
    k11 = _pad1d(keep1, 0.0)
    k21 = _pad1d(keep2, 0.0)

    xpad = jnp.concatenate([x, jnp.zeros((NPAD - N, D), _f32)])

    degparts = _deg_kernel(dst1, ew1, k11, k21).reshape(NC * 3, NPAD)
    dinv = _tc_prep(degparts)

    fm0r = jnp.ones((1, D), _f32)
    fm1r = fm1.reshape(1, D)
    fm2r = fm2.reshape(1, D)
    b1r = b1.reshape(1, D)
    b2r = b2.reshape(1, D)

    outs = []
    for keep1d_v, fmr, vi in ((ones1, fm0r, 0), (k11, fm1r, 1), (k21, fm2r, 2)):
        dv = dinv[vi]
        p1 = _prop_kernel(xpad, src1, dst1, ew1, keep1d_v, dv)
        h = _tc_dense(p1, fmr, W1, b1r, relu=True)
        p2 = _prop_kernel(h, src1, dst1, ew1, keep1d_v, dv)
        z = _tc_dense(p2, fm0r, W2, b2r, relu=False)
        outs.append(z[:N])

    return tuple(outs)


# speedup vs baseline: 1.1445x; 1.1445x over previous
import functools

import jax
import jax.numpy as jnp
from jax import lax
from jax.experimental import pallas as pl
from jax.experimental.pallas import tpu as pltpu
from jax.experimental.pallas import tpu_sc as plsc

N = 10000
E = 320000
D = 128

NC = 2
NS = 16
L = 16
NW = NC * NS

K = 80
CPT = 128
CH = NW * CPT
EPAD = CH * K
NBUF = 4

NPAD = 10240
NACC = 10112
RPTA = NACC // NS
RPT = NPAD // NS

KC = 512
CPTC = EPAD // (NW * KC)

_mesh = plsc.VectorSubcoreMesh(
    core_axis_name="c", subcore_axis_name="s", num_cores=NC, num_subcores=NS)
_params = pltpu.CompilerParams(needs_layout_passes=False)

_i32 = jnp.int32
_f32 = jnp.float32


def _deg_body(dst1, ew1, k11, k21, deg_out,
              h0, h1, h2, dst_v, ew_v, k1_v, k2_v, red_v, acc_v, stage_sh):
    c_ax = lax.axis_index("c")
    s_ax = lax.axis_index("s")
    wid = c_ax * NS + s_ax

    z16 = jnp.zeros((L,), _f32)

    def zero_body(i, _):
        h0[pl.ds(i * L, L)] = z16
        h1[pl.ds(i * L, L)] = z16
        h2[pl.ds(i * L, L)] = z16
        return 0
    lax.fori_loop(0, NPAD // L, zero_body, 0)

    def chunk_body(i, _):
        base = (wid * CPT + i) * K
        pltpu.sync_copy(dst1.at[pl.ds(base, K)], dst_v)
        pltpu.sync_copy(ew1.at[pl.ds(base, K)], ew_v)
        pltpu.sync_copy(k11.at[pl.ds(base, K)], k1_v)
        pltpu.sync_copy(k21.at[pl.ds(base, K)], k2_v)

        def grp(g, _):
            sl = pl.ds(g * L, L)
            d16 = dst_v[sl]
            w16 = ew_v[sl]
            plsc.addupdate_scatter(h0, [d16], w16)
            plsc.addupdate_scatter(h1, [d16], w16 * k1_v[sl])
            plsc.addupdate_scatter(h2, [d16], w16 * k2_v[sl])
            return 0
        lax.fori_loop(0, K // L, grp, 0)
        return 0
    lax.fori_loop(0, CPT, chunk_body, 0)

    pltpu.sync_copy(h0, stage_sh.at[pl.ds((s_ax * 3 + 0) * NPAD, NPAD)])
    pltpu.sync_copy(h1, stage_sh.at[pl.ds((s_ax * 3 + 1) * NPAD, NPAD)])
    pltpu.sync_copy(h2, stage_sh.at[pl.ds((s_ax * 3 + 2) * NPAD, NPAD)])
    plsc.subcore_barrier()

    for v in range(3):
        pltpu.sync_copy(stage_sh.at[pl.ds(v * NPAD + s_ax * RPT, RPT)], acc_v)

        def red_t(t, _):
            pltpu.sync_copy(
                stage_sh.at[pl.ds((t * 3 + v) * NPAD + s_ax * RPT, RPT)], red_v)

            def addg(g, _):
                sl = pl.ds(g * L, L)
                acc_v[sl] = acc_v[sl] + red_v[sl]
                return 0
            lax.fori_loop(0, RPT // L, addg, 0)
            return 0
        lax.fori_loop(1, NS, red_t, 0)
        pltpu.sync_copy(
            acc_v,
            deg_out.at[pl.ds((c_ax * 3 + v) * NPAD + s_ax * RPT, RPT)])


_deg_kernel = functools.partial(
    pl.kernel,
    out_type=jax.ShapeDtypeStruct((NC * 3 * NPAD,), _f32),
    mesh=_mesh,
    compiler_params=_params,
    scratch_types=[
        pltpu.VMEM((NPAD,), _f32),
        pltpu.VMEM((NPAD,), _f32),
        pltpu.VMEM((NPAD,), _f32),
        pltpu.VMEM((K,), _i32),
        pltpu.VMEM((K,), _f32),
        pltpu.VMEM((K,), _f32),
        pltpu.VMEM((K,), _f32),
        pltpu.VMEM((RPT,), _f32),
        pltpu.VMEM((RPT,), _f32),
        pltpu.VMEM_SHARED((NS * 3 * NPAD,), _f32),
    ],
)(_deg_body)


def _coef_body(src1, ew1, k11, k21, dv0_hbm, dv1_hbm, dv2_hbm, c_out,
               d0, d1, d2, src_v, ew_v, k1_v, k2_v, c0_v, c1_v, c2_v):
    c_ax = lax.axis_index("c")
    s_ax = lax.axis_index("s")
    wid = c_ax * NS + s_ax

    pltpu.sync_copy(dv0_hbm, d0)
    pltpu.sync_copy(dv1_hbm, d1)
    pltpu.sync_copy(dv2_hbm, d2)

    def chunk_body(i, _):
        base = (wid * CPTC + i) * KC
        pltpu.sync_copy(src1.at[pl.ds(base, KC)], src_v)
        pltpu.sync_copy(ew1.at[pl.ds(base, KC)], ew_v)
        pltpu.sync_copy(k11.at[pl.ds(base, KC)], k1_v)
        pltpu.sync_copy(k21.at[pl.ds(base, KC)], k2_v)

        def grp(g, _):
            sl = pl.ds(g * L, L)
            s16 = src_v[sl]
            w16 = ew_v[sl]
            c0_v[sl] = w16 * plsc.load_gather(d0, [s16])
            c1_v[sl] = w16 * k1_v[sl] * plsc.load_gather(d1, [s16])
            c2_v[sl] = w16 * k2_v[sl] * plsc.load_gather(d2, [s16])
            return 0
        lax.fori_loop(0, KC // L, grp, 0)

        pltpu.sync_copy(c0_v, c_out.at[pl.ds(0 * EPAD + base, KC)])
        pltpu.sync_copy(c1_v, c_out.at[pl.ds(1 * EPAD + base, KC)])
        pltpu.sync_copy(c2_v, c_out.at[pl.ds(2 * EPAD + base, KC)])
        return 0
    lax.fori_loop(0, CPTC, chunk_body, 0)


_coef_kernel = functools.partial(
    pl.kernel,
    out_type=jax.ShapeDtypeStruct((3 * EPAD,), _f32),
    mesh=_mesh,
    compiler_params=_params,
    scratch_types=(
        [pltpu.VMEM((NPAD,), _f32) for _ in range(3)]
        + [pltpu.VMEM((KC,), _i32)]
        + [pltpu.VMEM((KC,), _f32) for _ in range(6)]
    ),
)(_coef_body)


def _prop_body(table, src1, dst1, c1, dinv_hbm, part_out, *bufs):
    srcs = bufs[0:4]
    dsts = bufs[4:8]
    cs = bufs[8:12]
    rows = bufs[12:16]
    dinvw = bufs[16]
    acc_sh = bufs[17]
    gsem = bufs[18:22]
    esem = bufs[22:26]
    ssem = bufs[26:30]

    c_ax = lax.axis_index("c")
    s_ax = lax.axis_index("s")
    wid = c_ax * NS + s_ax

    z16 = jnp.zeros((L,), _f32)
    zi16 = jnp.zeros((L,), _i32)

    def zrow(k, _):
        for jj in range(D // L):
            rows[0][k, pl.ds(jj * L, L)] = z16
        return 0
    lax.fori_loop(0, K, zrow, 0)
    nfull = RPTA // K
    ntail = RPTA - nfull * K
    for q in range(nfull):
        pltpu.sync_copy(rows[0], acc_sh.at[pl.ds(s_ax * RPTA + q * K, K)])
    pltpu.sync_copy(rows[0].at[pl.ds(0, ntail)],
                    acc_sh.at[pl.ds(s_ax * RPTA + nfull * K, ntail)])
    plsc.subcore_barrier()

    cbase = wid * CPT

    def edge_dma(chunk, j):
        eb = (cbase + chunk) * K
        pltpu.async_copy(src1.at[pl.ds(eb, K)], srcs[j], esem[j])
        pltpu.async_copy(dst1.at[pl.ds(eb, K)], dsts[j], esem[j])
        pltpu.async_copy(c1.at[pl.ds(eb, K)], cs[j], esem[j])

    def edge_wait(j):
        pltpu.make_async_copy(src1.at[pl.ds(0, K)], srcs[j], esem[j]).wait()
        pltpu.make_async_copy(dst1.at[pl.ds(0, K)], dsts[j], esem[j]).wait()
        pltpu.make_async_copy(c1.at[pl.ds(0, K)], cs[j], esem[j]).wait()

    def gather(j):
        pltpu.async_copy(table.at[srcs[j]], rows[j], gsem[j])

    def gather_wait(j):
        pltpu.make_async_copy(table.at[srcs[j]], rows[j], gsem[j]).wait()

    def scatter(j):
        pltpu.async_copy(rows[j], acc_sh.at[dsts[j]], ssem[j], add=True)

    def scatter_wait(j):
        pltpu.make_async_copy(rows[j], acc_sh.at[dsts[j]], ssem[j]).wait()

    def compute(j):
        def scale(k, _):
            cb = plsc.load_gather(cs[j], [zi16 + k])
            for jj in range(D // L):
                sl = pl.ds(jj * L, L)
                rows[j][k, sl] = rows[j][k, sl] * cb
            return 0
        lax.fori_loop(0, K, scale, 0)

    def half(g, j, skip_swait):
        j1 = (j + 1) % NBUF
        j2 = (j + 2) % NBUF
        if isinstance(g, int):
            n2 = min(g + 2, CPT - 1)
        else:
            n2 = jnp.minimum(g + 2, CPT - 1)
        gather_wait(j)
        edge_wait(j1)
        gather(j1)
        if not skip_swait:
            scatter_wait(j2)
        edge_dma(n2, j2)
        compute(j)
        scatter(j)

    edge_dma(0, 0)
    edge_dma(1, 1)
    edge_wait(0)
    gather(0)
    half(0, 0, True)
    half(1, 1, True)
    half(2, 2, False)
    half(3, 3, False)

    def body(i, _):
        g = 4 * i
        half(g + 0, 0, False)
        half(g + 1, 1, False)
        half(g + 2, 2, False)
        half(g + 3, 3, False)
        return 0
    lax.fori_loop(1, CPT // NBUF, body, 0)

    gather_wait(0)
    edge_wait(1)
    scatter_wait(2)
    scatter_wait(3)
    plsc.subcore_barrier()

    for q in range(nfull + 1):
        base = s_ax * RPTA + q * K
        sz = K if q < nfull else ntail
        pltpu.sync_copy(dinv_hbm.at[pl.ds(base, sz)], dinvw.at[pl.ds(0, sz)])
        pltpu.sync_copy(acc_sh.at[pl.ds(base, sz)], rows[0].at[pl.ds(0, sz)])

        def wscale(k, _):
            db = plsc.load_gather(dinvw, [zi16 + k])
            for jj in range(D // L):
                sl = pl.ds(jj * L, L)
                rows[0][k, sl] = rows[0][k, sl] * db
            return 0
        lax.fori_loop(0, sz, wscale, 0)
        pltpu.sync_copy(rows[0].at[pl.ds(0, sz)],
                        part_out.at[pl.ds(c_ax * NACC + base, sz)])


_prop_kernel = functools.partial(
    pl.kernel,
    out_type=jax.ShapeDtypeStruct((NC * NACC, D), _f32),
    mesh=_mesh,
    compiler_params=_params,
    scratch_types=(
        [pltpu.VMEM((K,), _i32) for _ in range(4)]
        + [pltpu.VMEM((K,), _i32) for _ in range(4)]
        + [pltpu.VMEM((K,), _f32) for _ in range(4)]
        + [pltpu.VMEM((K, D), _f32) for _ in range(4)]
        + [pltpu.VMEM((K,), _f32)]
        + [pltpu.VMEM_SHARED((NACC, D), _f32)]
        + [pltpu.SemaphoreType.DMA for _ in range(12)]
    ),
)(_prop_body)


def _prep_body(deg_ref, dinv_ref):
    a = deg_ref[...]
    dinv_ref[...] = lax.rsqrt(a[0:3] + a[3:6] + 1e-9)


def _tc_prep(degparts):
    return pl.pallas_call(
        _prep_body,
        out_shape=jax.ShapeDtypeStruct((3, NPAD), _f32),
    )(degparts)


_BR = 1264


def _dense_body(p0_ref, p1_ref, fm_ref, w_ref, b_ref, o_ref, *, relu):
    a = p0_ref[...] + p1_ref[...]
    a = a * fm_ref[...]
    o = jnp.dot(a, w_ref[...], preferred_element_type=_f32) + b_ref[...]
    if relu:
        o = jnp.maximum(o, 0.0)
    o_ref[...] = o


def _tc_dense(parts, fm, W, b, relu):
    body = functools.partial(_dense_body, relu=relu)
    nb = NACC // _BR
    return pl.pallas_call(
        body,
        grid=(nb,),
        in_specs=[
            pl.BlockSpec((_BR, D), lambda i: (i, 0)),
            pl.BlockSpec((_BR, D), lambda i, nb=nb: (nb + i, 0)),
            pl.BlockSpec((1, D), lambda i: (0, 0)),
            pl.BlockSpec((D, D), lambda i: (0, 0)),
            pl.BlockSpec((1, D), lambda i: (0, 0)),
        ],
        out_specs=pl.BlockSpec((_BR, D), lambda i: (i, 0)),
        out_shape=jax.ShapeDtypeStruct((NACC, D), _f32),
    )(parts, parts, fm, W, b)


def _pad1d(a, fill):
    pad = EPAD - E
    return jnp.concatenate([a, jnp.full((pad,), fill, a.dtype)])


def kernel(x, edge_index, edge_weight, W1, b1, W2, b2):
    k = jax.random.key(42)
    k1, k2, k3, k4 = jax.random.split(k, 4)
    keep1 = jax.random.bernoulli(k1, 0.7, (E,)).astype(_f32)
    keep2 = jax.random.bernoulli(k2, 0.7, (E,)).astype(_f32)
    fm1 = jax.random.bernoulli(k3, 0.7, (D,)).astype(_f32)
    fm2 = jax.random.bernoulli(k4, 0.7, (D,)).astype(_f32)

    src1 = _pad1d(edge_index[0], 0)
    dst1 = _pad1d(edge_index[1], N)
    ew1 = _pad1d(edge_weight, 0.0)
    k11 = _pad1d(keep1, 0.0)
    k21 = _pad1d(keep2, 0.0)

    xpad = jnp.concatenate([x, jnp.zeros((NACC - N, D), _f32)])

    degparts = _deg_kernel(dst1, ew1, k11, k21).reshape(NC * 3, NPAD)
    dinv = _tc_prep(degparts)
    call = _coef_kernel(src1, ew1, k11, k21, dinv[0], dinv[1], dinv[2])

    fm0r = jnp.ones((1, D), _f32)
    b1r = b1.reshape(1, D)
    b2r = b2.reshape(1, D)

    outs = []
    for vi, fmr in ((0, fm0r), (1, fm1.reshape(1, D)), (2, fm2.reshape(1, D))):
        cv = call[vi * EPAD:(vi + 1) * EPAD]
        dv = dinv[vi]
        p1 = _prop_kernel(xpad, src1, dst1, cv, dv)
        h = _tc_dense(p1, fmr, W1, b1r, relu=True)
        p2 = _prop_kernel(h, src1, dst1, cv, dv)
        z = _tc_dense(p2, fm0r, W2, b2r, relu=False)
        outs.append(z[:N])

    return tuple(outs)


# speedup vs baseline: 2.9088x; 2.5415x over previous
import functools

import jax
import jax.numpy as jnp
from jax import lax
from jax.experimental import pallas as pl
from jax.experimental.pallas import tpu as pltpu
from jax.experimental.pallas import tpu_sc as plsc

N = 10000
E = 320000
D = 128

NC = 2
NS = 16
L = 16
NW = NC * NS

K = 80
CPT = 128
CH = NW * CPT
EPAD = CH * K
NBUF = 4

NPAD = 10240
NACC = 10112
RPTA = NACC // NS
RPT = NPAD // NS

KC = 512
CPTC = EPAD // (NW * KC)

_mesh = plsc.VectorSubcoreMesh(
    core_axis_name="c", subcore_axis_name="s", num_cores=NC, num_subcores=NS)
_params = pltpu.CompilerParams(needs_layout_passes=False)

_i32 = jnp.int32
_f32 = jnp.float32


def _deg_body(dst1, ew1, k11, k21, deg_out,
              h0, h1, h2, dst_v, ew_v, k1_v, k2_v, red_v, acc_v, stage_sh):
    c_ax = lax.axis_index("c")
    s_ax = lax.axis_index("s")
    wid = c_ax * NS + s_ax

    z16 = jnp.zeros((L,), _f32)

    def zero_body(i, _):
        h0[pl.ds(i * L, L)] = z16
        h1[pl.ds(i * L, L)] = z16
        h2[pl.ds(i * L, L)] = z16
        return 0
    lax.fori_loop(0, NPAD // L, zero_body, 0)

    def chunk_body(i, _):
        base = (wid * CPT + i) * K
        pltpu.sync_copy(dst1.at[pl.ds(base, K)], dst_v)
        pltpu.sync_copy(ew1.at[pl.ds(base, K)], ew_v)
        pltpu.sync_copy(k11.at[pl.ds(base, K)], k1_v)
        pltpu.sync_copy(k21.at[pl.ds(base, K)], k2_v)

        def grp(g, _):
            sl = pl.ds(g * L, L)
            d16 = dst_v[sl]
            w16 = ew_v[sl]
            plsc.addupdate_scatter(h0, [d16], w16)
            plsc.addupdate_scatter(h1, [d16], w16 * k1_v[sl])
            plsc.addupdate_scatter(h2, [d16], w16 * k2_v[sl])
            return 0
        lax.fori_loop(0, K // L, grp, 0)
        return 0
    lax.fori_loop(0, CPT, chunk_body, 0)

    pltpu.sync_copy(h0, stage_sh.at[pl.ds((s_ax * 3 + 0) * NPAD, NPAD)])
    pltpu.sync_copy(h1, stage_sh.at[pl.ds((s_ax * 3 + 1) * NPAD, NPAD)])
    pltpu.sync_copy(h2, stage_sh.at[pl.ds((s_ax * 3 + 2) * NPAD, NPAD)])
    plsc.subcore_barrier()

    for v in range(3):
        pltpu.sync_copy(stage_sh.at[pl.ds(v * NPAD + s_ax * RPT, RPT)], acc_v)

        def red_t(t, _):
            pltpu.sync_copy(
                stage_sh.at[pl.ds((t * 3 + v) * NPAD + s_ax * RPT, RPT)], red_v)

            def addg(g, _):
                sl = pl.ds(g * L, L)
                acc_v[sl] = acc_v[sl] + red_v[sl]
                return 0
            lax.fori_loop(0, RPT // L, addg, 0)
            return 0
        lax.fori_loop(1, NS, red_t, 0)
        pltpu.sync_copy(
            acc_v,
            deg_out.at[pl.ds((c_ax * 3 + v) * NPAD + s_ax * RPT, RPT)])


_deg_kernel = functools.partial(
    pl.kernel,
    out_type=jax.ShapeDtypeStruct((NC * 3 * NPAD,), _f32),
    mesh=_mesh,
    compiler_params=_params,
    scratch_types=[
        pltpu.VMEM((NPAD,), _f32),
        pltpu.VMEM((NPAD,), _f32),
        pltpu.VMEM((NPAD,), _f32),
        pltpu.VMEM((K,), _i32),
        pltpu.VMEM((K,), _f32),
        pltpu.VMEM((K,), _f32),
        pltpu.VMEM((K,), _f32),
        pltpu.VMEM((RPT,), _f32),
        pltpu.VMEM((RPT,), _f32),
        pltpu.VMEM_SHARED((NS * 3 * NPAD,), _f32),
    ],
)(_deg_body)


def _coef_body(src1, ew1, k11, k21, dv0_hbm, dv1_hbm, dv2_hbm, c_out,
               d0, d1, d2, src_v, ew_v, k1_v, k2_v, c0_v, c1_v, c2_v):
    c_ax = lax.axis_index("c")
    s_ax = lax.axis_index("s")
    wid = c_ax * NS + s_ax

    pltpu.sync_copy(dv0_hbm, d0)
    pltpu.sync_copy(dv1_hbm, d1)
    pltpu.sync_copy(dv2_hbm, d2)

    def chunk_body(i, _):
        base = (wid * CPTC + i) * KC
        pltpu.sync_copy(src1.at[pl.ds(base, KC)], src_v)
        pltpu.sync_copy(ew1.at[pl.ds(base, KC)], ew_v)
        pltpu.sync_copy(k11.at[pl.ds(base, KC)], k1_v)
        pltpu.sync_copy(k21.at[pl.ds(base, KC)], k2_v)

        def grp(g, _):
            sl = pl.ds(g * L, L)
            s16 = src_v[sl]
            w16 = ew_v[sl]
            c0_v[sl] = w16 * plsc.load_gather(d0, [s16])
            c1_v[sl] = w16 * k1_v[sl] * plsc.load_gather(d1, [s16])
            c2_v[sl] = w16 * k2_v[sl] * plsc.load_gather(d2, [s16])
            return 0
        lax.fori_loop(0, KC // L, grp, 0)

        pltpu.sync_copy(c0_v, c_out.at[pl.ds(0 * EPAD + base, KC)])
        pltpu.sync_copy(c1_v, c_out.at[pl.ds(1 * EPAD + base, KC)])
        pltpu.sync_copy(c2_v, c_out.at[pl.ds(2 * EPAD + base, KC)])
        return 0
    lax.fori_loop(0, CPTC, chunk_body, 0)


_coef_kernel = functools.partial(
    pl.kernel,
    out_type=jax.ShapeDtypeStruct((3 * EPAD,), _f32),
    mesh=_mesh,
    compiler_params=_params,
    scratch_types=(
        [pltpu.VMEM((NPAD,), _f32) for _ in range(3)]
        + [pltpu.VMEM((KC,), _i32)]
        + [pltpu.VMEM((KC,), _f32) for _ in range(6)]
    ),
)(_coef_body)


def _prop_body(table, src1, dst1, c1, dinv_hbm, part_out, *bufs):
    srcs = bufs[0:4]
    dsts = bufs[4:8]
    cs = bufs[8:12]
    rows = bufs[12:16]
    dinvw = bufs[16]
    acc_sh = bufs[17]
    gsem = bufs[18:22]
    esem = bufs[22:26]
    ssem = bufs[26:30]

    c_ax = lax.axis_index("c")
    s_ax = lax.axis_index("s")
    wid = c_ax * NS + s_ax

    z16 = jnp.zeros((L,), _f32)
    zi16 = jnp.zeros((L,), _i32)

    def zrow(k, _):
        for jj in range(D // L):
            rows[0][k, pl.ds(jj * L, L)] = z16
        return 0
    lax.fori_loop(0, K, zrow, 0)
    nfull = RPTA // K
    ntail = RPTA - nfull * K
    for q in range(nfull):
        pltpu.sync_copy(rows[0], acc_sh.at[pl.ds(s_ax * RPTA + q * K, K)])
    pltpu.sync_copy(rows[0].at[pl.ds(0, ntail)],
                    acc_sh.at[pl.ds(s_ax * RPTA + nfull * K, ntail)])
    plsc.subcore_barrier()

    cbase = wid * CPT

    def edge_dma(chunk, j):
        eb = (cbase + chunk) * K
        pltpu.async_copy(src1.at[pl.ds(eb, K)], srcs[j], esem[j])
        pltpu.async_copy(dst1.at[pl.ds(eb, K)], dsts[j], esem[j])
        pltpu.async_copy(c1.at[pl.ds(eb, K)], cs[j], esem[j])

    def edge_wait(j):
        pltpu.make_async_copy(src1.at[pl.ds(0, K)], srcs[j], esem[j]).wait()
        pltpu.make_async_copy(dst1.at[pl.ds(0, K)], dsts[j], esem[j]).wait()
        pltpu.make_async_copy(c1.at[pl.ds(0, K)], cs[j], esem[j]).wait()

    def gather(j):
        pltpu.async_copy(table.at[srcs[j]], rows[j], gsem[j])

    def gather_wait(j):
        pltpu.make_async_copy(table.at[srcs[j]], rows[j], gsem[j]).wait()

    def scatter(j):
        pltpu.async_copy(rows[j], acc_sh.at[dsts[j]], ssem[j], add=True)

    def scatter_wait(j):
        pltpu.make_async_copy(rows[j], acc_sh.at[dsts[j]], ssem[j]).wait()

    def compute(j):
        def scale(k, _):
            cb = plsc.load_gather(cs[j], [zi16 + k])
            for jj in range(D // L):
                sl = pl.ds(jj * L, L)
                rows[j][k, sl] = rows[j][k, sl] * cb
            return 0
        lax.fori_loop(0, K, scale, 0)

    def half(g, j, skip_swait):
        j1 = (j + 1) % NBUF
        j2 = (j + 2) % NBUF
        if isinstance(g, int):
            n2 = min(g + 2, CPT - 1)
        else:
            n2 = jnp.minimum(g + 2, CPT - 1)
        gather_wait(j)
        edge_wait(j1)
        gather(j1)
        if not skip_swait:
            scatter_wait(j2)
        edge_dma(n2, j2)
        compute(j)
        scatter(j)

    edge_dma(0, 0)
    edge_dma(1, 1)
    edge_wait(0)
    gather(0)
    half(0, 0, True)
    half(1, 1, True)
    half(2, 2, False)
    half(3, 3, False)

    def body(i, _):
        g = 4 * i
        half(g + 0, 0, False)
        half(g + 1, 1, False)
        half(g + 2, 2, False)
        half(g + 3, 3, False)
        return 0
    lax.fori_loop(1, CPT // NBUF, body, 0)

    gather_wait(0)
    edge_wait(1)
    scatter_wait(2)
    scatter_wait(3)
    plsc.subcore_barrier()

    for q in range(nfull + 1):
        base = s_ax * RPTA + q * K
        sz = K if q < nfull else ntail
        pltpu.sync_copy(dinv_hbm.at[pl.ds(base, sz)], dinvw.at[pl.ds(0, sz)])
        pltpu.sync_copy(acc_sh.at[pl.ds(base, sz)], rows[0].at[pl.ds(0, sz)])

        def wscale(k, _):
            db = plsc.load_gather(dinvw, [zi16 + k])
            for jj in range(D // L):
                sl = pl.ds(jj * L, L)
                rows[0][k, sl] = rows[0][k, sl] * db
            return 0
        lax.fori_loop(0, sz, wscale, 0)
        pltpu.sync_copy(rows[0].at[pl.ds(0, sz)],
                        part_out.at[pl.ds(c_ax * NACC + base, sz)])


_prop_kernel = functools.partial(
    pl.kernel,
    out_type=jax.ShapeDtypeStruct((NC * NACC, D), _f32),
    mesh=_mesh,
    compiler_params=_params,
    scratch_types=(
        [pltpu.VMEM((K,), _i32) for _ in range(4)]
        + [pltpu.VMEM((K,), _i32) for _ in range(4)]
        + [pltpu.VMEM((K,), _f32) for _ in range(4)]
        + [pltpu.VMEM((K, D), _f32) for _ in range(4)]
        + [pltpu.VMEM((K,), _f32)]
        + [pltpu.VMEM_SHARED((NACC, D), _f32)]
        + [pltpu.SemaphoreType.DMA for _ in range(12)]
    ),
)(_prop_body)


def _prep_body(deg_ref, dinv_ref):
    a = deg_ref[...]
    dinv_ref[...] = lax.rsqrt(a[0:3] + a[3:6] + 1e-9)


def _tc_prep(degparts):
    return pl.pallas_call(
        _prep_body,
        out_shape=jax.ShapeDtypeStruct((3, NPAD), _f32),
    )(degparts)


_BR = 1264


def _dense_body(p0_ref, p1_ref, fm_ref, w_ref, b_ref, o_ref, *, relu):
    a = p0_ref[...] + p1_ref[...]
    a = a * fm_ref[...]
    o = jnp.dot(a, w_ref[...], preferred_element_type=_f32) + b_ref[...]
    if relu:
        o = jnp.maximum(o, 0.0)
    o_ref[...] = o


def _tc_dense(parts, fm, W, b, relu):
    body = functools.partial(_dense_body, relu=relu)
    nb = NACC // _BR
    return pl.pallas_call(
        body,
        grid=(nb,),
        in_specs=[
            pl.BlockSpec((_BR, D), lambda i: (i, 0)),
            pl.BlockSpec((_BR, D), lambda i, nb=nb: (nb + i, 0)),
            pl.BlockSpec((1, D), lambda i: (0, 0)),
            pl.BlockSpec((D, D), lambda i: (0, 0)),
            pl.BlockSpec((1, D), lambda i: (0, 0)),
        ],
        out_specs=pl.BlockSpec((_BR, D), lambda i: (i, 0)),
        out_shape=jax.ShapeDtypeStruct((NACC, D), _f32),
    )(parts, parts, fm, W, b)


def _pad1d(a, fill):
    pad = EPAD - E
    return jnp.concatenate([a, jnp.full((pad,), fill, a.dtype)])


def _pad1d_spread(a, base, mod):
    pad = EPAD - E
    tail = base + (jnp.arange(pad, dtype=a.dtype) % mod)
    return jnp.concatenate([a, tail])


def kernel(x, edge_index, edge_weight, W1, b1, W2, b2):
    k = jax.random.key(42)
    k1, k2, k3, k4 = jax.random.split(k, 4)
    keep1 = jax.random.bernoulli(k1, 0.7, (E,)).astype(_f32)
    keep2 = jax.random.bernoulli(k2, 0.7, (E,)).astype(_f32)
    fm1 = jax.random.bernoulli(k3, 0.7, (D,)).astype(_f32)
    fm2 = jax.random.bernoulli(k4, 0.7, (D,)).astype(_f32)

    src1 = _pad1d_spread(edge_index[0], 0, N)
    dst1 = _pad1d_spread(edge_index[1], N, NACC - N)
    ew1 = _pad1d(edge_weight, 0.0)
    k11 = _pad1d(keep1, 0.0)
    k21 = _pad1d(keep2, 0.0)

    xpad = jnp.concatenate([x, jnp.zeros((NACC - N, D), _f32)])

    degparts = _deg_kernel(dst1, ew1, k11, k21).reshape(NC * 3, NPAD)
    dinv = _tc_prep(degparts)
    call = _coef_kernel(src1, ew1, k11, k21, dinv[0], dinv[1], dinv[2])

    fm0r = jnp.ones((1, D), _f32)
    b1r = b1.reshape(1, D)
    b2r = b2.reshape(1, D)

    outs = []
    for vi, fmr in ((0, fm0r), (1, fm1.reshape(1, D)), (2, fm2.reshape(1, D))):
        cv = call[vi * EPAD:(vi + 1) * EPAD]
        dv = dinv[vi]
        p1 = _prop_kernel(xpad, src1, dst1, cv, dv)
        h = _tc_dense(p1, fmr, W1, b1r, relu=True)
        p2 = _prop_kernel(h, src1, dst1, cv, dv)
        z = _tc_dense(p2, fm0r, W2, b2r, relu=False)
        outs.append(z[:N])

    return tuple(outs)


# speedup vs baseline: 3.4598x; 1.1894x over previous
import functools

import jax
import jax.numpy as jnp
from jax import lax
from jax.experimental import pallas as pl
from jax.experimental.pallas import tpu as pltpu
from jax.experimental.pallas import tpu_sc as plsc

N = 10000
E = 320000
D = 128

NC = 2
NS = 16
L = 16
NW = NC * NS

K = 80
CPT = 128
CH = NW * CPT
EPAD = CH * K
NBUF = 4

NPAD = 10240
NACC = 10112
RPTA = NACC // NS
RPT = NPAD // NS

KC = 2048
CPTC = EPAD // (NW * KC)
KD = 1280
CPTD = EPAD // (NW * KD)

_mesh = plsc.VectorSubcoreMesh(
    core_axis_name="c", subcore_axis_name="s", num_cores=NC, num_subcores=NS)
_params = pltpu.CompilerParams(needs_layout_passes=False)

_i32 = jnp.int32
_f32 = jnp.float32


def _deg_body(dst1, ew1, k11, k21, deg_out,
              h0, h1, h2, dst_v, ew_v, k1_v, k2_v, red_v, acc_v, stage_sh):
    c_ax = lax.axis_index("c")
    s_ax = lax.axis_index("s")
    wid = c_ax * NS + s_ax

    z16 = jnp.zeros((L,), _f32)

    def zero_body(i, _):
        h0[pl.ds(i * L, L)] = z16
        h1[pl.ds(i * L, L)] = z16
        h2[pl.ds(i * L, L)] = z16
        return 0
    lax.fori_loop(0, NPAD // L, zero_body, 0)

    def chunk_body(i, _):
        base = (wid * CPTD + i) * KD
        pltpu.sync_copy(dst1.at[pl.ds(base, KD)], dst_v)
        pltpu.sync_copy(ew1.at[pl.ds(base, KD)], ew_v)
        pltpu.sync_copy(k11.at[pl.ds(base, KD)], k1_v)
        pltpu.sync_copy(k21.at[pl.ds(base, KD)], k2_v)

        def grp(g, _):
            sl = pl.ds(g * L, L)
            d16 = dst_v[sl]
            w16 = ew_v[sl]
            plsc.addupdate_scatter(h0, [d16], w16)
            plsc.addupdate_scatter(h1, [d16], w16 * k1_v[sl])
            plsc.addupdate_scatter(h2, [d16], w16 * k2_v[sl])
            return 0
        lax.fori_loop(0, KD // L, grp, 0)
        return 0
    lax.fori_loop(0, CPTD, chunk_body, 0)

    pltpu.sync_copy(h0, stage_sh.at[pl.ds((s_ax * 3 + 0) * NPAD, NPAD)])
    pltpu.sync_copy(h1, stage_sh.at[pl.ds((s_ax * 3 + 1) * NPAD, NPAD)])
    pltpu.sync_copy(h2, stage_sh.at[pl.ds((s_ax * 3 + 2) * NPAD, NPAD)])
    plsc.subcore_barrier()

    for v in range(3):
        pltpu.sync_copy(stage_sh.at[pl.ds(v * NPAD + s_ax * RPT, RPT)], acc_v)

        def red_t(t, _):
            pltpu.sync_copy(
                stage_sh.at[pl.ds((t * 3 + v) * NPAD + s_ax * RPT, RPT)], red_v)

            def addg(g, _):
                sl = pl.ds(g * L, L)
                acc_v[sl] = acc_v[sl] + red_v[sl]
                return 0
            lax.fori_loop(0, RPT // L, addg, 0)
            return 0
        lax.fori_loop(1, NS, red_t, 0)
        pltpu.sync_copy(
            acc_v,
            deg_out.at[pl.ds((c_ax * 3 + v) * NPAD + s_ax * RPT, RPT)])


_deg_kernel = functools.partial(
    pl.kernel,
    out_type=jax.ShapeDtypeStruct((NC * 3 * NPAD,), _f32),
    mesh=_mesh,
    compiler_params=_params,
    scratch_types=[
        pltpu.VMEM((NPAD,), _f32),
        pltpu.VMEM((NPAD,), _f32),
        pltpu.VMEM((NPAD,), _f32),
        pltpu.VMEM((KD,), _i32),
        pltpu.VMEM((KD,), _f32),
        pltpu.VMEM((KD,), _f32),
        pltpu.VMEM((KD,), _f32),
        pltpu.VMEM((RPT,), _f32),
        pltpu.VMEM((RPT,), _f32),
        pltpu.VMEM_SHARED((NS * 3 * NPAD,), _f32),
    ],
)(_deg_body)


def _coef_body(src1, ew1, k11, k21, dv0_hbm, dv1_hbm, dv2_hbm, c_out,
               d0, d1, d2, src_v, ew_v, k1_v, k2_v, c0_v, c1_v, c2_v):
    c_ax = lax.axis_index("c")
    s_ax = lax.axis_index("s")
    wid = c_ax * NS + s_ax

    pltpu.sync_copy(dv0_hbm, d0)
    pltpu.sync_copy(dv1_hbm, d1)
    pltpu.sync_copy(dv2_hbm, d2)

    def chunk_body(i, _):
        base = (wid * CPTC + i) * KC
        pltpu.sync_copy(src1.at[pl.ds(base, KC)], src_v)
        pltpu.sync_copy(ew1.at[pl.ds(base, KC)], ew_v)
        pltpu.sync_copy(k11.at[pl.ds(base, KC)], k1_v)
        pltpu.sync_copy(k21.at[pl.ds(base, KC)], k2_v)

        def grp(g, _):
            sl = pl.ds(g * L, L)
            s16 = src_v[sl]
            w16 = ew_v[sl]
            c0_v[sl] = w16 * plsc.load_gather(d0, [s16])
            c1_v[sl] = w16 * k1_v[sl] * plsc.load_gather(d1, [s16])
            c2_v[sl] = w16 * k2_v[sl] * plsc.load_gather(d2, [s16])
            return 0
        lax.fori_loop(0, KC // L, grp, 0)

        pltpu.sync_copy(c0_v, c_out.at[pl.ds(0 * EPAD + base, KC)])
        pltpu.sync_copy(c1_v, c_out.at[pl.ds(1 * EPAD + base, KC)])
        pltpu.sync_copy(c2_v, c_out.at[pl.ds(2 * EPAD + base, KC)])
        return 0
    lax.fori_loop(0, CPTC, chunk_body, 0)


_coef_kernel = functools.partial(
    pl.kernel,
    out_type=jax.ShapeDtypeStruct((3 * EPAD,), _f32),
    mesh=_mesh,
    compiler_params=_params,
    scratch_types=(
        [pltpu.VMEM((NPAD,), _f32) for _ in range(3)]
        + [pltpu.VMEM((KC,), _i32)]
        + [pltpu.VMEM((KC,), _f32) for _ in range(6)]
    ),
)(_coef_body)


def _prop_body(table, src1, dst1, c1, dinv_hbm, part_out, *bufs):
    srcs = bufs[0:4]
    dsts = bufs[4:8]
    cs = bufs[8:12]
    rows = bufs[12:16]
    dinvw = bufs[16]
    acc_sh = bufs[17]
    gsem = bufs[18:22]
    esem = bufs[22:26]
    ssem = bufs[26:30]

    c_ax = lax.axis_index("c")
    s_ax = lax.axis_index("s")
    wid = c_ax * NS + s_ax

    z16 = jnp.zeros((L,), _f32)
    zi16 = jnp.zeros((L,), _i32)

    def zrow(k, _):
        for jj in range(D // L):
            rows[0][k, pl.ds(jj * L, L)] = z16
        return 0
    lax.fori_loop(0, K, zrow, 0)
    nfull = RPTA // K
    ntail = RPTA - nfull * K
    for q in range(nfull):
        pltpu.sync_copy(rows[0], acc_sh.at[pl.ds(s_ax * RPTA + q * K, K)])
    pltpu.sync_copy(rows[0].at[pl.ds(0, ntail)],
                    acc_sh.at[pl.ds(s_ax * RPTA + nfull * K, ntail)])
    plsc.subcore_barrier()

    cbase = wid * CPT

    def edge_dma(chunk, j):
        eb = (cbase + chunk) * K
        pltpu.async_copy(src1.at[pl.ds(eb, K)], srcs[j], esem[j])
        pltpu.async_copy(dst1.at[pl.ds(eb, K)], dsts[j], esem[j])
        pltpu.async_copy(c1.at[pl.ds(eb, K)], cs[j], esem[j])

    def edge_wait(j):
        pltpu.make_async_copy(src1.at[pl.ds(0, K)], srcs[j], esem[j]).wait()
        pltpu.make_async_copy(dst1.at[pl.ds(0, K)], dsts[j], esem[j]).wait()
        pltpu.make_async_copy(c1.at[pl.ds(0, K)], cs[j], esem[j]).wait()

    def gather(j):
        pltpu.async_copy(table.at[srcs[j]], rows[j], gsem[j])

    def gather_wait(j):
        pltpu.make_async_copy(table.at[srcs[j]], rows[j], gsem[j]).wait()

    def scatter(j):
        pltpu.async_copy(rows[j], acc_sh.at[dsts[j]], ssem[j], add=True)

    def scatter_wait(j):
        pltpu.make_async_copy(rows[j], acc_sh.at[dsts[j]], ssem[j]).wait()

    def compute(j):
        def scale(k, _):
            cb = plsc.load_gather(cs[j], [zi16 + k])
            for jj in range(D // L):
                sl = pl.ds(jj * L, L)
                rows[j][k, sl] = rows[j][k, sl] * cb
            return 0
        lax.fori_loop(0, K, scale, 0)

    def half(g, j, skip_swait):
        j1 = (j + 1) % NBUF
        j2 = (j + 2) % NBUF
        if isinstance(g, int):
            n2 = min(g + 2, CPT - 1)
        else:
            n2 = jnp.minimum(g + 2, CPT - 1)
        gather_wait(j)
        edge_wait(j1)
        gather(j1)
        if not skip_swait:
            scatter_wait(j2)
        edge_dma(n2, j2)
        compute(j)
        scatter(j)

    edge_dma(0, 0)
    edge_dma(1, 1)
    edge_wait(0)
    gather(0)
    half(0, 0, True)
    half(1, 1, True)
    half(2, 2, False)
    half(3, 3, False)

    def body(i, _):
        g = 4 * i
        half(g + 0, 0, False)
        half(g + 1, 1, False)
        half(g + 2, 2, False)
        half(g + 3, 3, False)
        return 0
    lax.fori_loop(1, CPT // NBUF, body, 0)

    gather_wait(0)
    edge_wait(1)
    scatter_wait(2)
    scatter_wait(3)
    plsc.subcore_barrier()

    for q in range(nfull + 1):
        base = s_ax * RPTA + q * K
        sz = K if q < nfull else ntail
        pltpu.sync_copy(dinv_hbm.at[pl.ds(base, sz)], dinvw.at[pl.ds(0, sz)])
        pltpu.sync_copy(acc_sh.at[pl.ds(base, sz)], rows[0].at[pl.ds(0, sz)])

        def wscale(k, _):
            db = plsc.load_gather(dinvw, [zi16 + k])
            for jj in range(D // L):
                sl = pl.ds(jj * L, L)
                rows[0][k, sl] = rows[0][k, sl] * db
            return 0
        lax.fori_loop(0, sz, wscale, 0)
        pltpu.sync_copy(rows[0].at[pl.ds(0, sz)],
                        part_out.at[pl.ds(c_ax * NACC + base, sz)])


_prop_kernel = functools.partial(
    pl.kernel,
    out_type=jax.ShapeDtypeStruct((NC * NACC, D), _f32),
    mesh=_mesh,
    compiler_params=_params,
    scratch_types=(
        [pltpu.VMEM((K,), _i32) for _ in range(4)]
        + [pltpu.VMEM((K,), _i32) for _ in range(4)]
        + [pltpu.VMEM((K,), _f32) for _ in range(4)]
        + [pltpu.VMEM((K, D), _f32) for _ in range(4)]
        + [pltpu.VMEM((K,), _f32)]
        + [pltpu.VMEM_SHARED((NACC, D), _f32)]
        + [pltpu.SemaphoreType.DMA for _ in range(12)]
    ),
)(_prop_body)


def _prep_body(deg_ref, dinv_ref):
    a = deg_ref[...]
    dinv_ref[...] = lax.rsqrt(a[0:3] + a[3:6] + 1e-9)


def _tc_prep(degparts):
    return pl.pallas_call(
        _prep_body,
        out_shape=jax.ShapeDtypeStruct((3, NPAD), _f32),
    )(degparts)


_BR = 1264


def _dense_body(p0_ref, p1_ref, fm_ref, w_ref, b_ref, o_ref, *, relu):
    a = p0_ref[...] + p1_ref[...]
    a = a * fm_ref[...]
    o = jnp.dot(a, w_ref[...], preferred_element_type=_f32) + b_ref[...]
    if relu:
        o = jnp.maximum(o, 0.0)
    o_ref[...] = o


def _tc_dense(parts, fm, W, b, relu):
    body = functools.partial(_dense_body, relu=relu)
    nb = NACC // _BR
    return pl.pallas_call(
        body,
        grid=(nb,),
        in_specs=[
            pl.BlockSpec((_BR, D), lambda i: (i, 0)),
            pl.BlockSpec((_BR, D), lambda i, nb=nb: (nb + i, 0)),
            pl.BlockSpec((1, D), lambda i: (0, 0)),
            pl.BlockSpec((D, D), lambda i: (0, 0)),
            pl.BlockSpec((1, D), lambda i: (0, 0)),
        ],
        out_specs=pl.BlockSpec((_BR, D), lambda i: (i, 0)),
        out_shape=jax.ShapeDtypeStruct((NACC, D), _f32),
    )(parts, parts, fm, W, b)


def _pad1d(a, fill):
    pad = EPAD - E
    return jnp.concatenate([a, jnp.full((pad,), fill, a.dtype)])


def _pad1d_spread(a, base, mod):
    pad = EPAD - E
    tail = base + (jnp.arange(pad, dtype=a.dtype) % mod)
    return jnp.concatenate([a, tail])


def kernel(x, edge_index, edge_weight, W1, b1, W2, b2):
    k = jax.random.key(42)
    k1, k2, k3, k4 = jax.random.split(k, 4)
    keep1 = jax.random.bernoulli(k1, 0.7, (E,)).astype(_f32)
    keep2 = jax.random.bernoulli(k2, 0.7, (E,)).astype(_f32)
    fm1 = jax.random.bernoulli(k3, 0.7, (D,)).astype(_f32)
    fm2 = jax.random.bernoulli(k4, 0.7, (D,)).astype(_f32)

    src1 = _pad1d_spread(edge_index[0], 0, N)
    dst1 = _pad1d_spread(edge_index[1], N, NACC - N)
    ew1 = _pad1d(edge_weight, 0.0)
    k11 = _pad1d(keep1, 0.0)
    k21 = _pad1d(keep2, 0.0)

    xpad = jnp.concatenate([x, jnp.zeros((NACC - N, D), _f32)])

    degparts = _deg_kernel(dst1, ew1, k11, k21).reshape(NC * 3, NPAD)
    dinv = _tc_prep(degparts)
    call = _coef_kernel(src1, ew1, k11, k21, dinv[0], dinv[1], dinv[2])

    fm0r = jnp.ones((1, D), _f32)
    b1r = b1.reshape(1, D)
    b2r = b2.reshape(1, D)

    outs = []
    for vi, fmr in ((0, fm0r), (1, fm1.reshape(1, D)), (2, fm2.reshape(1, D))):
        cv = call[vi * EPAD:(vi + 1) * EPAD]
        dv = dinv[vi]
        p1 = _prop_kernel(xpad, src1, dst1, cv, dv)
        h = _tc_dense(p1, fmr, W1, b1r, relu=True)
        p2 = _prop_kernel(h, src1, dst1, cv, dv)
        z = _tc_dense(p2, fm0r, W2, b2r, relu=False)
        outs.append(z[:N])

    return tuple(outs)
